# Initial kernel scaffold; baseline (speedup 1.0000x reference)
import jax
import jax.numpy as jnp
from jax.experimental import pallas as pl


def kernel(x, table):
    raise NotImplementedError("write your pallas kernel here")


# speedup vs baseline: 1.8751x; 1.8751x over previous
import jax
import jax.numpy as jnp
from jax import lax
from jax.experimental import pallas as pl
from jax.experimental.pallas import tpu as pltpu
from jax.experimental.pallas import tpu_sc as plsc

_D = 64
_SUB = 128
_CHUNK = 640
_SPC = _CHUNK // _SUB
_NC, _NS = 2, 16
_NW = _NC * _NS


def _emb_body(idx_hbm, table_hbm, out_hbm, idx_v, rows0, rows1, sem0, sem1):
    wid = lax.axis_index("s") * _NC + lax.axis_index("c")
    ipw_rows = idx_hbm.shape[0] // _NW
    n_chunks = ipw_rows // _SPC
    row0 = wid * ipw_rows
    out0 = wid * ipw_rows * _SUB

    pltpu.sync_copy(idx_hbm.at[pl.ds(row0, ipw_rows)], idx_v)

    def fire(chunk, buf, sem):
        for j in range(_SPC):
            pltpu.make_async_copy(
                table_hbm.at[idx_v.at[chunk * _SPC + j]],
                buf.at[pl.ds(j * _SUB, _SUB)],
                sem,
            ).start()

    def drain(buf, sem):
        pltpu.make_async_copy(table_hbm.at[pl.ds(0, _CHUNK)], buf, sem).wait()

    bufs = (rows0, rows1)
    sems = (sem0, sem1)

    fire(0, rows0, sem0)

    def body(g2, carry):
        g = g2 * 2
        for b in range(2):
            cur = g + b
            nxt = cur + 1

            @pl.when(nxt < n_chunks)
            def _():
                fire(nxt, bufs[1 - b], sems[1 - b])

            drain(bufs[b], sems[b])
            pltpu.sync_copy(bufs[b],
                            out_hbm.at[pl.ds(out0 + cur * _CHUNK, _CHUNK)])
        return carry

    lax.fori_loop(0, n_chunks // 2, body, 0)


def kernel(x, table):
    bsz, hist = x.shape
    n = bsz * hist
    assert n % (_NW * _CHUNK) == 0 and (n // (_NW * _CHUNK)) % 2 == 0
    idx = x.reshape(n // _SUB, _SUB).astype(jnp.int32)
    ipw_rows = n // (_NW * _SUB)
    mesh = plsc.VectorSubcoreMesh(core_axis_name="c", subcore_axis_name="s")
    out = pl.kernel(
        _emb_body,
        out_type=jax.ShapeDtypeStruct((n, _D), jnp.float32),
        mesh=mesh,
        scratch_types=[
            pltpu.VMEM((ipw_rows, _SUB), jnp.int32),
            pltpu.VMEM((_CHUNK, _D), jnp.float32),
            pltpu.VMEM((_CHUNK, _D), jnp.float32),
            pltpu.SemaphoreType.DMA,
            pltpu.SemaphoreType.DMA,
        ],
        compiler_params=pltpu.CompilerParams(use_tc_tiling_on_sc=False),
    )(idx, table)
    return out.reshape(bsz, hist, _D)


# speedup vs baseline: 2.4893x; 1.3275x over previous
import jax
import jax.numpy as jnp
from jax import lax
from jax.experimental import pallas as pl
from jax.experimental.pallas import tpu as pltpu
from jax.experimental.pallas import tpu_sc as plsc

_D = 64
_DP = 128
_SUB = 128
_BLK = 256
_SPC = _BLK // _SUB
_NC, _NS = 2, 16
_NW = _NC * _NS
_TBC = 8192
_OBB = 2048


def _tc_pad_transpose(table_t):
    v = table_t.shape[1]

    def body(in_ref, out_ref):
        out_ref[:, :_D] = in_ref[...].T

    grid = (v + _TBC - 1) // _TBC
    return pl.pallas_call(
        body,
        grid=(grid,),
        in_specs=[pl.BlockSpec((_D, _TBC), lambda i: (0, i))],
        out_specs=pl.BlockSpec((_TBC, _DP), lambda i: (i, 0)),
        out_shape=jax.ShapeDtypeStruct((v, _DP), jnp.float32),
    )(table_t)


def _tc_out_transpose(g3):
    hist, bsz, _ = g3.shape

    def body(in_ref, out_ref):
        out_ref[...] = in_ref[0][:, :_D].T

    return pl.pallas_call(
        body,
        grid=(hist, bsz // _OBB),
        in_specs=[pl.BlockSpec((1, _OBB, _DP), lambda t, j: (t, j, 0))],
        out_specs=pl.BlockSpec((_D, _OBB), lambda t, j: (t, j)),
        out_shape=jax.ShapeDtypeStruct((hist * _D, bsz), jnp.float32),
    )(g3)


def _emb_body(idx_hbm, table_hbm, out_hbm,
              idx_v, buf0, buf1, gsem0, gsem1, wsem0, wsem1):
    hist, bsz = idx_hbm.shape
    b_per_w = bsz // _NW
    hpb = b_per_w // _BLK
    n_chunks = hist * hpb
    wid = lax.axis_index("s") * _NC + lax.axis_index("c")
    b0 = wid * b_per_w

    bufs = (buf0, buf1)
    gsems = (gsem0, gsem1)
    wsems = (wsem0, wsem1)

    pltpu.sync_copy(idx_hbm.at[:, pl.ds(b0, b_per_w)], idx_v)

    def fire(k, slot):
        t = k // hpb
        h = k % hpb
        for j in range(_SPC):
            pltpu.make_async_copy(
                table_hbm.at[idx_v.at[t, pl.ds(h * _BLK + j * _SUB, _SUB)]],
                bufs[slot].at[pl.ds(j * _SUB, _SUB)],
                gsems[slot],
            ).start()

    def gdrain(slot):
        pltpu.make_async_copy(
            table_hbm.at[pl.ds(0, _BLK)], bufs[slot], gsems[slot]).wait()

    def wdrain(slot):
        pltpu.make_async_copy(
            table_hbm.at[pl.ds(0, _BLK)],
            bufs[slot],
            wsems[slot],
        ).wait()

    fire(0, 0)

    def pair_body(k2, carry):
        for s in range(2):
            k = k2 * 2 + s
            t = k // hpb
            bcol = b0 + (k % hpb) * _BLK

            gdrain(s)

            @pl.when(k + 1 < n_chunks)
            def _():
                @pl.when(k > 0)
                def _():
                    wdrain(1 - s)
                fire(k + 1, 1 - s)

            pltpu.make_async_copy(
                bufs[s],
                out_hbm.at[pl.ds(t * bsz + bcol, _BLK)],
                wsems[s],
            ).start()
        return carry

    lax.fori_loop(0, n_chunks // 2, pair_body, 0)
    wdrain(0)
    wdrain(1)


def kernel(x, table):
    bsz, hist = x.shape
    n_emb, d = table.shape
    assert d == _D and bsz % (_NW * _BLK) == 0
    idx_t = x.astype(jnp.int32).T
    table_pad = _tc_pad_transpose(table.T)
    mesh = plsc.VectorSubcoreMesh(core_axis_name="c", subcore_axis_name="s")
    g = pl.kernel(
        _emb_body,
        out_type=jax.ShapeDtypeStruct((hist * bsz, _DP), jnp.float32),
        mesh=mesh,
        scratch_types=[
            pltpu.VMEM((hist, bsz // _NW), jnp.int32),
            pltpu.VMEM((_BLK, _DP), jnp.float32),
            pltpu.VMEM((_BLK, _DP), jnp.float32),
            pltpu.SemaphoreType.DMA,
            pltpu.SemaphoreType.DMA,
            pltpu.SemaphoreType.DMA,
            pltpu.SemaphoreType.DMA,
        ],
        compiler_params=pltpu.CompilerParams(use_tc_tiling_on_sc=True),
    )(idx_t, table_pad)
    out2 = _tc_out_transpose(g.reshape(hist, bsz, _DP))
    return jnp.transpose(out2.reshape(hist, _D, bsz), (2, 0, 1))


# speedup vs baseline: 2.8530x; 1.1461x over previous
import jax
import jax.numpy as jnp
from jax import lax
from jax.experimental import pallas as pl
from jax.experimental.pallas import tpu as pltpu
from jax.experimental.pallas import tpu_sc as plsc

_D = 64
_DP = 128
_SUB = 128
_BLK = 256
_SPC = _BLK // _SUB
_NC, _NS = 2, 16
_NW = _NC * _NS
_TBC = 16384
_OBB = 4096


def _tc_pad_transpose(table_t):
    v = table_t.shape[1]

    def body(in_ref, out_ref):
        out_ref[:, :_D] = in_ref[...].T

    grid = (v + _TBC - 1) // _TBC
    return pl.pallas_call(
        body,
        grid=(grid,),
        in_specs=[pl.BlockSpec((_D, _TBC), lambda i: (0, i))],
        out_specs=pl.BlockSpec((_TBC, _DP), lambda i: (i, 0)),
        out_shape=jax.ShapeDtypeStruct((v, _DP), jnp.float32),
    )(table_t)


def _tc_out_transpose(g3):
    hist, bsz, _ = g3.shape

    def body(in_ref, out_ref):
        out_ref[...] = in_ref[0][:, :_D].T

    return pl.pallas_call(
        body,
        grid=(hist, bsz // _OBB),
        in_specs=[pl.BlockSpec((1, _OBB, _DP), lambda t, j: (t, j, 0))],
        out_specs=pl.BlockSpec((_D, _OBB), lambda t, j: (t, j)),
        out_shape=jax.ShapeDtypeStruct((hist * _D, bsz), jnp.float32),
    )(g3)


def _emb_body(idx_hbm, table_hbm, out_hbm,
              idx_v, buf0, buf1, gsem0, gsem1, wsem0, wsem1):
    hist, bsz = idx_hbm.shape
    b_per_w = bsz // _NW
    hpb = b_per_w // _BLK
    n_chunks = hist * hpb
    wid = lax.axis_index("s") * _NC + lax.axis_index("c")
    b0 = wid * b_per_w

    bufs = (buf0, buf1)
    gsems = (gsem0, gsem1)
    wsems = (wsem0, wsem1)

    pltpu.sync_copy(idx_hbm.at[:, pl.ds(b0, b_per_w)], idx_v)

    def fire(k, slot):
        t = k // hpb
        h = k % hpb
        for j in range(_SPC):
            pltpu.make_async_copy(
                table_hbm.at[idx_v.at[t, pl.ds(h * _BLK + j * _SUB, _SUB)]],
                bufs[slot].at[pl.ds(j * _SUB, _SUB)],
                gsems[slot],
            ).start()

    def gdrain(slot):
        pltpu.make_async_copy(
            table_hbm.at[pl.ds(0, _BLK)], bufs[slot], gsems[slot]).wait()

    def wdrain(slot):
        pltpu.make_async_copy(
            table_hbm.at[pl.ds(0, _BLK)],
            bufs[slot],
            wsems[slot],
        ).wait()

    fire(0, 0)

    def pair_body(k2, carry):
        for s in range(2):
            k = k2 * 2 + s
            t = k // hpb
            bcol = b0 + (k % hpb) * _BLK

            gdrain(s)

            @pl.when(k + 1 < n_chunks)
            def _():
                @pl.when(k > 0)
                def _():
                    wdrain(1 - s)
                fire(k + 1, 1 - s)

            pltpu.make_async_copy(
                bufs[s],
                out_hbm.at[pl.ds(t * bsz + bcol, _BLK)],
                wsems[s],
            ).start()
        return carry

    lax.fori_loop(0, n_chunks // 2, pair_body, 0)
    wdrain(0)
    wdrain(1)


def kernel(x, table):
    bsz, hist = x.shape
    n_emb, d = table.shape
    assert d == _D and bsz % (_NW * _BLK) == 0
    idx_t = x.astype(jnp.int32).T
    table_pad = _tc_pad_transpose(table.T)
    mesh = plsc.VectorSubcoreMesh(core_axis_name="c", subcore_axis_name="s")
    g = pl.kernel(
        _emb_body,
        out_type=jax.ShapeDtypeStruct((hist * bsz, _DP), jnp.float32),
        mesh=mesh,
        scratch_types=[
            pltpu.VMEM((hist, bsz // _NW), jnp.int32),
            pltpu.VMEM((_BLK, _DP), jnp.float32),
            pltpu.VMEM((_BLK, _DP), jnp.float32),
            pltpu.SemaphoreType.DMA,
            pltpu.SemaphoreType.DMA,
            pltpu.SemaphoreType.DMA,
            pltpu.SemaphoreType.DMA,
        ],
        compiler_params=pltpu.CompilerParams(use_tc_tiling_on_sc=True),
    )(idx_t, table_pad)
    out2 = _tc_out_transpose(g.reshape(hist, bsz, _DP))
    return jnp.transpose(out2.reshape(hist, _D, bsz), (2, 0, 1))


# speedup vs baseline: 3.1051x; 1.0883x over previous
import jax
import jax.numpy as jnp
from jax import lax
from jax.experimental import pallas as pl
from jax.experimental.pallas import tpu as pltpu
from jax.experimental.pallas import tpu_sc as plsc

_D = 64
_DP = 128
_SUB = 128
_BLK = 256
_SPC = _BLK // _SUB
_NC, _NS = 2, 16
_NW = _NC * _NS
_TBC = 32768
_OBB = 8192


def _tc_pad_transpose(table_t):
    v = table_t.shape[1]

    def body(in_ref, out_ref):
        out_ref[:, :_D] = in_ref[...].T

    grid = (v + _TBC - 1) // _TBC
    return pl.pallas_call(
        body,
        grid=(grid,),
        in_specs=[pl.BlockSpec((_D, _TBC), lambda i: (0, i))],
        out_specs=pl.BlockSpec((_TBC, _DP), lambda i: (i, 0)),
        out_shape=jax.ShapeDtypeStruct((v, _DP), jnp.float32),
    )(table_t)


def _tc_out_transpose(g3):
    hist, bsz, _ = g3.shape

    def body(in_ref, out_ref):
        out_ref[...] = in_ref[0][:, :_D].T

    return pl.pallas_call(
        body,
        grid=(hist, bsz // _OBB),
        in_specs=[pl.BlockSpec((1, _OBB, _DP), lambda t, j: (t, j, 0))],
        out_specs=pl.BlockSpec((_D, _OBB), lambda t, j: (t, j)),
        out_shape=jax.ShapeDtypeStruct((hist * _D, bsz), jnp.float32),
    )(g3)


def _emb_body(idx_hbm, table_hbm, out_hbm,
              idx_v, buf0, buf1, gsem0, gsem1, wsem0, wsem1):
    hist, bsz = idx_hbm.shape
    b_per_w = bsz // _NW
    hpb = b_per_w // _BLK
    n_chunks = hist * hpb
    wid = lax.axis_index("s") * _NC + lax.axis_index("c")
    b0 = wid * b_per_w

    bufs = (buf0, buf1)
    gsems = (gsem0, gsem1)
    wsems = (wsem0, wsem1)

    pltpu.sync_copy(idx_hbm.at[:, pl.ds(b0, b_per_w)], idx_v)

    def fire(k, slot):
        t = k // hpb
        h = k % hpb
        for j in range(_SPC):
            pltpu.make_async_copy(
                table_hbm.at[idx_v.at[t, pl.ds(h * _BLK + j * _SUB, _SUB)]],
                bufs[slot].at[pl.ds(j * _SUB, _SUB)],
                gsems[slot],
            ).start()

    def gdrain(slot):
        pltpu.make_async_copy(
            table_hbm.at[pl.ds(0, _BLK)], bufs[slot], gsems[slot]).wait()

    def wdrain(slot):
        pltpu.make_async_copy(
            table_hbm.at[pl.ds(0, _BLK)],
            bufs[slot],
            wsems[slot],
        ).wait()

    fire(0, 0)

    def pair_body(k2, carry):
        for s in range(2):
            k = k2 * 2 + s
            t = k // hpb
            bcol = b0 + (k % hpb) * _BLK

            gdrain(s)

            @pl.when(k + 1 < n_chunks)
            def _():
                @pl.when(k > 0)
                def _():
                    wdrain(1 - s)
                fire(k + 1, 1 - s)

            pltpu.make_async_copy(
                bufs[s],
                out_hbm.at[pl.ds(t * bsz + bcol, _BLK)],
                wsems[s],
            ).start()
        return carry

    lax.fori_loop(0, n_chunks // 2, pair_body, 0)
    wdrain(0)
    wdrain(1)


def kernel(x, table):
    bsz, hist = x.shape
    n_emb, d = table.shape
    assert d == _D and bsz % (_NW * _BLK) == 0
    idx_t = x.astype(jnp.int32).T
    table_pad = _tc_pad_transpose(table.T)
    mesh = plsc.VectorSubcoreMesh(core_axis_name="c", subcore_axis_name="s")
    g = pl.kernel(
        _emb_body,
        out_type=jax.ShapeDtypeStruct((hist * bsz, _DP), jnp.float32),
        mesh=mesh,
        scratch_types=[
            pltpu.VMEM((hist, bsz // _NW), jnp.int32),
            pltpu.VMEM((_BLK, _DP), jnp.float32),
            pltpu.VMEM((_BLK, _DP), jnp.float32),
            pltpu.SemaphoreType.DMA,
            pltpu.SemaphoreType.DMA,
            pltpu.SemaphoreType.DMA,
            pltpu.SemaphoreType.DMA,
        ],
        compiler_params=pltpu.CompilerParams(use_tc_tiling_on_sc=True),
    )(idx_t, table_pad)
    out2 = _tc_out_transpose(g.reshape(hist, bsz, _DP))
    return jnp.transpose(out2.reshape(hist, _D, bsz), (2, 0, 1))


# speedup vs baseline: 3.1535x; 1.0156x over previous
import jax
import jax.numpy as jnp
from jax import lax
from jax.experimental import pallas as pl
from jax.experimental.pallas import tpu as pltpu
from jax.experimental.pallas import tpu_sc as plsc

_D = 64
_DP = 128
_SUB = 128
_BLK = 256
_SPC = _BLK // _SUB
_NC, _NS = 2, 16
_NW = _NC * _NS
_TBC = 32768
_OBB = 16384


def _tc_pad_transpose(table_t):
    v = table_t.shape[1]

    def body(in_ref, out_ref):
        out_ref[:, :_D] = in_ref[...].T

    grid = (v + _TBC - 1) // _TBC
    return pl.pallas_call(
        body,
        grid=(grid,),
        in_specs=[pl.BlockSpec((_D, _TBC), lambda i: (0, i))],
        out_specs=pl.BlockSpec((_TBC, _DP), lambda i: (i, 0)),
        out_shape=jax.ShapeDtypeStruct((v, _DP), jnp.float32),
    )(table_t)


def _tc_out_transpose(g3):
    hist, bsz, _ = g3.shape

    def body(in_ref, out_ref):
        out_ref[...] = in_ref[0][:, :_D].T

    return pl.pallas_call(
        body,
        grid=(hist, bsz // _OBB),
        in_specs=[pl.BlockSpec((1, _OBB, _DP), lambda t, j: (t, j, 0))],
        out_specs=pl.BlockSpec((_D, _OBB), lambda t, j: (t, j)),
        out_shape=jax.ShapeDtypeStruct((hist * _D, bsz), jnp.float32),
    )(g3)


def _emb_body(idx_hbm, table_hbm, out_hbm,
              idx_v, buf0, buf1, gsem0, gsem1, wsem0, wsem1):
    hist, bsz = idx_hbm.shape
    b_per_w = bsz // _NW
    hpb = b_per_w // _BLK
    n_chunks = hist * hpb
    wid = lax.axis_index("s") * _NC + lax.axis_index("c")
    b0 = wid * b_per_w

    bufs = (buf0, buf1)
    gsems = (gsem0, gsem1)
    wsems = (wsem0, wsem1)

    pltpu.sync_copy(idx_hbm.at[:, pl.ds(b0, b_per_w)], idx_v)

    def fire(k, slot):
        t = k // hpb
        h = k % hpb
        for j in range(_SPC):
            pltpu.make_async_copy(
                table_hbm.at[idx_v.at[t, pl.ds(h * _BLK + j * _SUB, _SUB)]],
                bufs[slot].at[pl.ds(j * _SUB, _SUB)],
                gsems[slot],
            ).start()

    def gdrain(slot):
        pltpu.make_async_copy(
            table_hbm.at[pl.ds(0, _BLK)], bufs[slot], gsems[slot]).wait()

    def wdrain(slot):
        pltpu.make_async_copy(
            table_hbm.at[pl.ds(0, _BLK)],
            bufs[slot],
            wsems[slot],
        ).wait()

    fire(0, 0)

    def pair_body(k2, carry):
        for s in range(2):
            k = k2 * 2 + s
            t = k // hpb
            bcol = b0 + (k % hpb) * _BLK

            gdrain(s)

            @pl.when(k + 1 < n_chunks)
            def _():
                @pl.when(k > 0)
                def _():
                    wdrain(1 - s)
                fire(k + 1, 1 - s)

            pltpu.make_async_copy(
                bufs[s],
                out_hbm.at[pl.ds(t * bsz + bcol, _BLK)],
                wsems[s],
            ).start()
        return carry

    lax.fori_loop(0, n_chunks // 2, pair_body, 0)
    wdrain(0)
    wdrain(1)


def kernel(x, table):
    bsz, hist = x.shape
    n_emb, d = table.shape
    assert d == _D and bsz % (_NW * _BLK) == 0
    idx_t = x.astype(jnp.int32).T
    table_pad = _tc_pad_transpose(table.T)
    mesh = plsc.VectorSubcoreMesh(core_axis_name="c", subcore_axis_name="s")
    g = pl.kernel(
        _emb_body,
        out_type=jax.ShapeDtypeStruct((hist * bsz, _DP), jnp.float32),
        mesh=mesh,
        scratch_types=[
            pltpu.VMEM((hist, bsz // _NW), jnp.int32),
            pltpu.VMEM((_BLK, _DP), jnp.float32),
            pltpu.VMEM((_BLK, _DP), jnp.float32),
            pltpu.SemaphoreType.DMA,
            pltpu.SemaphoreType.DMA,
            pltpu.SemaphoreType.DMA,
            pltpu.SemaphoreType.DMA,
        ],
        compiler_params=pltpu.CompilerParams(use_tc_tiling_on_sc=True),
    )(idx_t, table_pad)
    out2 = _tc_out_transpose(g.reshape(hist, bsz, _DP))
    return jnp.transpose(out2.reshape(hist, _D, bsz), (2, 0, 1))
